# Initial kernel scaffold; baseline (speedup 1.0000x reference)
import jax
import jax.numpy as jnp
from jax import lax
from jax.experimental import pallas as pl
from jax.experimental.pallas import tpu as pltpu
from jax.experimental.pallas import tpu_sc as plsc

N = 100000
E = 3200000
NUM_ELEM = 5
D = 16
NRBF = 16
CODES_PER = 4
K = NUM_ELEM * CODES_PER
NPAIR = K * K
ANGSTROM_TO_BOHR = 1.8897259886
R_MIN = 0.5 * ANGSTROM_TO_BOHR
R_MAX = 7.56
RBF_CENTERS = jnp.linspace(R_MIN, R_MAX, NRBF, dtype=jnp.float32)
RBF_WIDTH = (R_MAX - R_MIN) / NRBF
COMMIT_W = 0.25

NC = 2
NS = 16
NW = NC * NS
EPW = E // NW
WIN = 2000
NWIN = EPW // WIN
CH = 125
NCHUNK = WIN // CH
G16 = WIN // 16

NPAD = 102400
TPB = NPAD // NS
BN = 5120
NB = NPAD // BN
BE = 6400
NEB = E // BE

_MESH = plsc.VectorSubcoreMesh(
    core_axis_name="c", subcore_axis_name="s", num_cores=NC, num_subcores=NS
)


def _p1_body(elem_hbm, tgt_hbm, out_hbm, elem_v, tgt_v, out_v):
    wid = lax.axis_index("s") * NC + lax.axis_index("c")
    pltpu.sync_copy(elem_hbm, elem_v)

    def win(w, carry):
        base = wid * EPW + w * WIN
        pltpu.sync_copy(tgt_hbm.at[pl.ds(base, WIN)], tgt_v)

        def grp(j, carry2):
            idx = tgt_v[pl.ds(j * 16, 16)]
            out_v[pl.ds(j * 16, 16)] = plsc.load_gather(elem_v, [idx])
            return carry2

        lax.fori_loop(0, G16, grp, 0)
        pltpu.sync_copy(out_v, out_hbm.at[pl.ds(base, WIN)])
        return carry

    lax.fori_loop(0, NWIN, win, 0)


_p1 = pl.kernel(
    _p1_body,
    out_type=jax.ShapeDtypeStruct((E,), jnp.int32),
    mesh=_MESH,
    scratch_types=[
        pltpu.VMEM((N,), jnp.int32),
        pltpu.VMEM((WIN,), jnp.int32),
        pltpu.VMEM((WIN,), jnp.int32),
    ],
)


def _p2_body(d_ref, et_ref, emb_ref, fw1_ref, fb1_ref, fw2_ref, fb2_ref, out_ref):
    d = d_ref[:]
    z = (d[:, None] - RBF_CENTERS[None, :]) / RBF_WIDTH
    rbf = jnp.exp(-0.5 * (z * z))
    h1 = jax.nn.silu(jnp.dot(rbf, fw1_ref[:], preferred_element_type=jnp.float32) + fb1_ref[:])
    wf = jnp.dot(h1, fw2_ref[:], preferred_element_type=jnp.float32) + fb2_ref[:]
    et = et_ref[:][:, None]
    e = emb_ref[:]
    row = jnp.where(
        et == 0,
        e[0][None, :],
        jnp.where(
            et == 1,
            e[1][None, :],
            jnp.where(et == 2, e[2][None, :], jnp.where(et == 3, e[3][None, :], e[4][None, :])),
        ),
    )
    out_ref[:] = row * wf


_p2 = pl.pallas_call(
    _p2_body,
    grid=(NEB,),
    in_specs=[
        pl.BlockSpec((BE,), lambda i: (i,)),
        pl.BlockSpec((BE,), lambda i: (i,)),
        pl.BlockSpec((NUM_ELEM, D), lambda i: (0, 0)),
        pl.BlockSpec((NRBF, D), lambda i: (0, 0)),
        pl.BlockSpec((1, D), lambda i: (0, 0)),
        pl.BlockSpec((D, D), lambda i: (0, 0)),
        pl.BlockSpec((1, D), lambda i: (0, 0)),
    ],
    out_specs=pl.BlockSpec((BE, D), lambda i: (i, 0)),
    out_shape=jax.ShapeDtypeStruct((E, D), jnp.float32),
)


def _p3_body(src2_hbm, msg_hbm, zer_hbm, agg_hbm, agg_s, idx_v, msg_v):
    c = lax.axis_index("c")
    s = lax.axis_index("s")
    wid = s * NC + c
    pltpu.sync_copy(zer_hbm, agg_s.at[pl.ds(s * TPB, TPB)])
    plsc.subcore_barrier()

    def win(w, carry):
        base = wid * EPW + w * WIN
        rb = wid * (EPW // CH) + w * NCHUNK
        pltpu.sync_copy(src2_hbm.at[pl.ds(rb, NCHUNK)], idx_v)
        pltpu.sync_copy(msg_hbm.at[pl.ds(base, WIN)], msg_v)
        for j in range(NCHUNK):
            pltpu.sync_copy(msg_v.at[pl.ds(j * CH, CH)], agg_s.at[idx_v.at[j]], add=True)
        return carry

    lax.fori_loop(0, NWIN, win, 0)
    plsc.subcore_barrier()
    pltpu.sync_copy(agg_s.at[pl.ds(s * TPB, TPB)], agg_hbm.at[c, pl.ds(s * TPB, TPB)])


_p3 = pl.kernel(
    _p3_body,
    out_type=jax.ShapeDtypeStruct((NC, NPAD, D), jnp.float32),
    mesh=_MESH,
    scratch_types=[
        pltpu.VMEM_SHARED((NPAD, D), jnp.float32),
        pltpu.VMEM((NCHUNK, CH), jnp.int32),
        pltpu.VMEM((WIN, D), jnp.float32),
    ],
)


def _p4_body(
    elem_ref, agg_ref, emb_ref, cb_ref, cbT_ref, mw1_ref, mb1_ref, mw2_ref, mb2_ref,
    atom_ref, loss_ref, t_ref,
):
    i = pl.program_id(0)
    ei = elem_ref[:][:, None]
    e = emb_ref[:]
    h0 = jnp.where(
        ei == 0,
        e[0][None, :],
        jnp.where(
            ei == 1,
            e[1][None, :],
            jnp.where(ei == 2, e[2][None, :], jnp.where(ei == 3, e[3][None, :], e[4][None, :])),
        ),
    )
    h = h0 + agg_ref[0] + agg_ref[1]
    cbT = cbT_ref[:]
    xc = jnp.dot(h, cbT, preferred_element_type=jnp.float32)
    hsq = jnp.sum(h * h, axis=1, keepdims=True)
    csq = jnp.sum(cbT * cbT, axis=0, keepdims=True)
    d2 = jnp.maximum(hsq + csq - 2.0 * xc, 0.0)
    code_elem = lax.broadcasted_iota(jnp.int32, (1, K), 1) // CODES_PER
    masked = jnp.where(ei == code_elem, d2, jnp.inf)
    m = jnp.min(masked, axis=1)
    jio = lax.broadcasted_iota(jnp.int32, (BN, K), 1)
    atom_ref[:] = jnp.min(jnp.where(masked == m[:, None], jio, K), axis=1)
    rows = i * BN + lax.broadcasted_iota(jnp.int32, (BN,), 0)
    partial = jnp.sum(jnp.where(rows < N, m, 0.0)) * (COMMIT_W / (N * D))

    @pl.when(i == 0)
    def _init():
        loss_ref[:] = jnp.zeros((8, 128), jnp.float32)
        cb = cb_ref[:]
        ii = lax.broadcasted_iota(jnp.int32, (NPAIR, K), 1)
        ridx = lax.broadcasted_iota(jnp.int32, (NPAIR, 1), 0)
        oh = (ii == ridx // K).astype(jnp.float32) + (ii == ridx % K).astype(jnp.float32)
        pair = jnp.dot(oh, cb, preferred_element_type=jnp.float32)
        hh = jax.nn.silu(jnp.dot(pair, mw1_ref[:], preferred_element_type=jnp.float32) + mb1_ref[:])
        t_ref[:] = jnp.dot(hh, mw2_ref[:], preferred_element_type=jnp.float32) + mb2_ref[:]

    loss_ref[:] = loss_ref[:] + partial


_p4 = pl.pallas_call(
    _p4_body,
    grid=(NB,),
    in_specs=[
        pl.BlockSpec((BN,), lambda i: (i,)),
        pl.BlockSpec((NC, BN, D), lambda i: (0, i, 0)),
        pl.BlockSpec((NUM_ELEM, D), lambda i: (0, 0)),
        pl.BlockSpec((K, D), lambda i: (0, 0)),
        pl.BlockSpec((D, K), lambda i: (0, 0)),
        pl.BlockSpec((D, D), lambda i: (0, 0)),
        pl.BlockSpec((1, D), lambda i: (0, 0)),
        pl.BlockSpec((D, 3), lambda i: (0, 0)),
        pl.BlockSpec((1, 3), lambda i: (0, 0)),
    ],
    out_specs=[
        pl.BlockSpec((BN,), lambda i: (i,)),
        pl.BlockSpec((8, 128), lambda i: (0, 0)),
        pl.BlockSpec((NPAIR, 3), lambda i: (0, 0)),
    ],
    out_shape=[
        jax.ShapeDtypeStruct((NPAD,), jnp.int32),
        jax.ShapeDtypeStruct((8, 128), jnp.float32),
        jax.ShapeDtypeStruct((NPAIR, 3), jnp.float32),
    ],
)


def _p5_body(src_hbm, tgt_hbm, atoms_hbm, t_hbm, out_hbm, atoms_v, t_v, src_v, tgt_v, out_v):
    wid = lax.axis_index("s") * NC + lax.axis_index("c")
    pltpu.sync_copy(atoms_hbm, atoms_v)
    pltpu.sync_copy(t_hbm, t_v)
    arange3 = jnp.arange(16, dtype=jnp.int32) * 3

    def win(w, carry):
        base = wid * EPW + w * WIN
        pltpu.sync_copy(src_hbm.at[pl.ds(base, WIN)], src_v)
        pltpu.sync_copy(tgt_hbm.at[pl.ds(base, WIN)], tgt_v)

        def grp(j, carry2):
            s16 = src_v[pl.ds(j * 16, 16)]
            t16 = tgt_v[pl.ds(j * 16, 16)]
            a = plsc.load_gather(atoms_v, [s16])
            b = plsc.load_gather(atoms_v, [t16])
            f3 = (a * K + b) * 3
            oi = arange3 + j * 48
            plsc.store_scatter(out_v, [oi], plsc.load_gather(t_v, [f3]))
            plsc.store_scatter(out_v, [oi + 1], plsc.load_gather(t_v, [f3 + 1]))
            plsc.store_scatter(out_v, [oi + 2], plsc.load_gather(t_v, [f3 + 2]))
            return carry2

        lax.fori_loop(0, G16, grp, 0)
        pltpu.sync_copy(out_v, out_hbm.at[pl.ds(base * 3, WIN * 3)])
        return carry

    lax.fori_loop(0, NWIN, win, 0)


_p5 = pl.kernel(
    _p5_body,
    out_type=jax.ShapeDtypeStruct((E * 3,), jnp.float32),
    mesh=_MESH,
    scratch_types=[
        pltpu.VMEM((NPAD,), jnp.int32),
        pltpu.VMEM((NPAIR * 3,), jnp.float32),
        pltpu.VMEM((WIN,), jnp.int32),
        pltpu.VMEM((WIN,), jnp.int32),
        pltpu.VMEM((WIN * 3,), jnp.float32),
    ],
)


def kernel(elem_idx, edge_index, distances, embed, codebook, fw1, fb1, fw2, fb2, mw1, mb1, mw2, mb2):
    src = edge_index[0]
    tgt = edge_index[1]
    elem_tgt = _p1(elem_idx, tgt)
    messages = _p2(distances, elem_tgt, embed, fw1, fb1.reshape(1, D), fw2, fb2.reshape(1, D))
    src2 = src.reshape(E // CH, CH)
    zeros = jnp.zeros((TPB, D), jnp.float32)
    agg = _p3(src2, messages, zeros)
    elem_pad = jnp.pad(elem_idx, (0, NPAD - N))
    atoms_pad, lossbuf, t_tab = _p4(
        elem_pad, agg, embed, codebook, codebook.T, mw1, mb1.reshape(1, D), mw2,
        mb2.reshape(1, 3),
    )
    params_flat = _p5(src, tgt, atoms_pad, t_tab.reshape(NPAIR * 3))
    return (params_flat.reshape(E, 3), atoms_pad[:N], lossbuf[0, 0])


# speedup vs baseline: 9.1638x; 9.1638x over previous
import jax
import jax.numpy as jnp
from jax import lax
from jax.experimental import pallas as pl
from jax.experimental.pallas import tpu as pltpu
from jax.experimental.pallas import tpu_sc as plsc

N = 100000
E = 3200000
NUM_ELEM = 5
D = 16
NRBF = 16
CODES_PER = 4
K = NUM_ELEM * CODES_PER
NPAIR = K * K
ANGSTROM_TO_BOHR = 1.8897259886
R_MIN = 0.5 * ANGSTROM_TO_BOHR
R_MAX = 7.56
RBF_CENTERS = jnp.linspace(R_MIN, R_MAX, NRBF, dtype=jnp.float32)
RBF_WIDTH = (R_MAX - R_MIN) / NRBF
COMMIT_W = 0.25

NC = 2
NS = 16
NW = NC * NS
EPW = E // NW
WIN = 2000
NWIN = EPW // WIN
CH = 125
G16 = WIN // 16
WIN3 = 1000
NWIN3 = EPW // WIN3
NCHUNK3 = WIN3 // CH

NPAD = 102400
TPB = NPAD // NS
BN = 5120
NB = NPAD // BN
BE = 5120
NEB = E // BE

_MESH = plsc.VectorSubcoreMesh(
    core_axis_name="c", subcore_axis_name="s", num_cores=NC, num_subcores=NS
)


def _p1_body(elem_hbm, tgt_hbm, out_hbm, elem_v, tgt_v, out_v):
    wid = lax.axis_index("s") * NC + lax.axis_index("c")
    pltpu.sync_copy(elem_hbm, elem_v)

    def win(w, carry):
        base = wid * EPW + w * WIN
        pltpu.sync_copy(tgt_hbm.at[pl.ds(base, WIN)], tgt_v)

        def grp(j, carry2):
            idx = tgt_v[pl.ds(j * 16, 16)]
            out_v[pl.ds(j * 16, 16)] = plsc.load_gather(elem_v, [idx])
            return carry2

        lax.fori_loop(0, G16, grp, 0)
        pltpu.sync_copy(out_v, out_hbm.at[pl.ds(base, WIN)])
        return carry

    lax.fori_loop(0, NWIN, win, 0)


_p1 = pl.kernel(
    _p1_body,
    out_type=jax.ShapeDtypeStruct((E,), jnp.int32),
    mesh=_MESH,
    compiler_params=pltpu.CompilerParams(needs_layout_passes=False, use_tc_tiling_on_sc=False),
    scratch_types=[
        pltpu.VMEM((N,), jnp.int32),
        pltpu.VMEM((WIN,), jnp.int32),
        pltpu.VMEM((WIN,), jnp.int32),
    ],
)


def _p2_body(d_ref, ctr_ref, et_ref, emb_ref, fw1_ref, fb1_ref, fw2_ref, fb2_ref, out_ref):
    d = d_ref[:]
    z = (d[:, None] - ctr_ref[:]) / RBF_WIDTH
    rbf = jnp.exp(-0.5 * (z * z))
    h1 = jax.nn.silu(jnp.dot(rbf, fw1_ref[:], preferred_element_type=jnp.float32) + fb1_ref[:])
    wf = jnp.dot(h1, fw2_ref[:], preferred_element_type=jnp.float32) + fb2_ref[:]
    et = et_ref[:][:, None]
    e = emb_ref[:]
    row = jnp.where(
        et == 0,
        e[0][None, :],
        jnp.where(
            et == 1,
            e[1][None, :],
            jnp.where(et == 2, e[2][None, :], jnp.where(et == 3, e[3][None, :], e[4][None, :])),
        ),
    )
    out_ref[:] = row * wf


_p2 = pl.pallas_call(
    _p2_body,
    grid=(NEB,),
    in_specs=[
        pl.BlockSpec((BE,), lambda i: (i,)),
        pl.BlockSpec((1, NRBF), lambda i: (0, 0)),
        pl.BlockSpec((BE,), lambda i: (i,)),
        pl.BlockSpec((NUM_ELEM, D), lambda i: (0, 0)),
        pl.BlockSpec((NRBF, D), lambda i: (0, 0)),
        pl.BlockSpec((1, D), lambda i: (0, 0)),
        pl.BlockSpec((D, D), lambda i: (0, 0)),
        pl.BlockSpec((1, D), lambda i: (0, 0)),
    ],
    out_specs=pl.BlockSpec((BE, D), lambda i: (i, 0)),
    out_shape=jax.ShapeDtypeStruct((E, D), jnp.float32),
)


def _p3_body(src2_hbm, msg_hbm, zer_hbm, agg_hbm, agg_s, idx_v, msg_v):
    c = lax.axis_index("c")
    s = lax.axis_index("s")
    wid = s * NC + c
    pltpu.sync_copy(zer_hbm, agg_s.at[pl.ds(s * TPB, TPB)])
    plsc.subcore_barrier()

    def win(w, carry):
        base = wid * EPW + w * WIN3
        rb = wid * (EPW // CH) + w * NCHUNK3
        pltpu.sync_copy(src2_hbm.at[pl.ds(rb, NCHUNK3)], idx_v)
        pltpu.sync_copy(msg_hbm.at[pl.ds(base, WIN3)], msg_v)
        for j in range(NCHUNK3):
            pltpu.sync_copy(msg_v.at[pl.ds(j * CH, CH)], agg_s.at[idx_v.at[j]], add=True)
        return carry

    lax.fori_loop(0, NWIN3, win, 0)
    plsc.subcore_barrier()
    pltpu.sync_copy(agg_s.at[pl.ds(s * TPB, TPB)], agg_hbm.at[c, pl.ds(s * TPB, TPB)])


_p3 = pl.kernel(
    _p3_body,
    out_type=jax.ShapeDtypeStruct((NC, NPAD, D), jnp.float32),
    mesh=_MESH,
    compiler_params=pltpu.CompilerParams(needs_layout_passes=False, use_tc_tiling_on_sc=False),
    scratch_types=[
        pltpu.VMEM_SHARED((NPAD, D), jnp.float32),
        pltpu.VMEM((NCHUNK3, CH), jnp.int32),
        pltpu.VMEM((WIN3, D), jnp.float32),
    ],
)


def _p4_body(
    elem_ref, agg_ref, emb_ref, cb_ref, cbT_ref, mw1_ref, mb1_ref, mw2_ref, mb2_ref,
    atom_ref, loss_ref, t_ref,
):
    i = pl.program_id(0)
    ei = elem_ref[:][:, None]
    e = emb_ref[:]
    h0 = jnp.where(
        ei == 0,
        e[0][None, :],
        jnp.where(
            ei == 1,
            e[1][None, :],
            jnp.where(ei == 2, e[2][None, :], jnp.where(ei == 3, e[3][None, :], e[4][None, :])),
        ),
    )
    h = h0 + agg_ref[0] + agg_ref[1]
    cbT = cbT_ref[:]
    xc = jnp.dot(h, cbT, preferred_element_type=jnp.float32)
    hsq = jnp.sum(h * h, axis=1, keepdims=True)
    csq = jnp.sum(cbT * cbT, axis=0, keepdims=True)
    d2 = jnp.maximum(hsq + csq - 2.0 * xc, 0.0)
    code_elem = lax.broadcasted_iota(jnp.int32, (1, K), 1) // CODES_PER
    masked = jnp.where(ei == code_elem, d2, jnp.inf)
    m = jnp.min(masked, axis=1)
    jio = lax.broadcasted_iota(jnp.int32, (BN, K), 1)
    atom_ref[:] = jnp.min(jnp.where(masked == m[:, None], jio, K), axis=1)
    rows = i * BN + lax.broadcasted_iota(jnp.int32, (BN,), 0)
    partial = jnp.sum(jnp.where(rows < N, m, 0.0)) * (COMMIT_W / (N * D))

    @pl.when(i == 0)
    def _init():
        loss_ref[:] = jnp.zeros((8, 128), jnp.float32)
        cb = cb_ref[:]
        ii = lax.broadcasted_iota(jnp.int32, (NPAIR, K), 1)
        ridx = lax.broadcasted_iota(jnp.int32, (NPAIR, 1), 0)
        oh = (ii == ridx // K).astype(jnp.float32) + (ii == ridx % K).astype(jnp.float32)
        pair = jnp.dot(oh, cb, preferred_element_type=jnp.float32)
        hh = jax.nn.silu(jnp.dot(pair, mw1_ref[:], preferred_element_type=jnp.float32) + mb1_ref[:])
        t_ref[:] = jnp.dot(hh, mw2_ref[:], preferred_element_type=jnp.float32) + mb2_ref[:]

    loss_ref[:] = loss_ref[:] + partial


_p4 = pl.pallas_call(
    _p4_body,
    grid=(NB,),
    in_specs=[
        pl.BlockSpec((BN,), lambda i: (i,)),
        pl.BlockSpec((NC, BN, D), lambda i: (0, i, 0)),
        pl.BlockSpec((NUM_ELEM, D), lambda i: (0, 0)),
        pl.BlockSpec((K, D), lambda i: (0, 0)),
        pl.BlockSpec((D, K), lambda i: (0, 0)),
        pl.BlockSpec((D, D), lambda i: (0, 0)),
        pl.BlockSpec((1, D), lambda i: (0, 0)),
        pl.BlockSpec((D, 3), lambda i: (0, 0)),
        pl.BlockSpec((1, 3), lambda i: (0, 0)),
    ],
    out_specs=[
        pl.BlockSpec((BN,), lambda i: (i,)),
        pl.BlockSpec((8, 128), lambda i: (0, 0)),
        pl.BlockSpec((NPAIR, 3), lambda i: (0, 0)),
    ],
    out_shape=[
        jax.ShapeDtypeStruct((NPAD,), jnp.int32),
        jax.ShapeDtypeStruct((8, 128), jnp.float32),
        jax.ShapeDtypeStruct((NPAIR, 3), jnp.float32),
    ],
)


def _p5_body(src_hbm, tgt_hbm, atoms_hbm, t_hbm, out_hbm, atoms_v, t_v, src_v, tgt_v, out_v):
    wid = lax.axis_index("s") * NC + lax.axis_index("c")
    pltpu.sync_copy(atoms_hbm, atoms_v)
    pltpu.sync_copy(t_hbm, t_v)
    arange3 = jnp.arange(16, dtype=jnp.int32) * 3

    def win(w, carry):
        base = wid * EPW + w * WIN
        pltpu.sync_copy(src_hbm.at[pl.ds(base, WIN)], src_v)
        pltpu.sync_copy(tgt_hbm.at[pl.ds(base, WIN)], tgt_v)

        def grp(j, carry2):
            s16 = src_v[pl.ds(j * 16, 16)]
            t16 = tgt_v[pl.ds(j * 16, 16)]
            a = plsc.load_gather(atoms_v, [s16])
            b = plsc.load_gather(atoms_v, [t16])
            f3 = (a * K + b) * 3
            oi = arange3 + j * 48
            plsc.store_scatter(out_v, [oi], plsc.load_gather(t_v, [f3]))
            plsc.store_scatter(out_v, [oi + 1], plsc.load_gather(t_v, [f3 + 1]))
            plsc.store_scatter(out_v, [oi + 2], plsc.load_gather(t_v, [f3 + 2]))
            return carry2

        lax.fori_loop(0, G16, grp, 0)
        pltpu.sync_copy(out_v, out_hbm.at[pl.ds(base * 3, WIN * 3)])
        return carry

    lax.fori_loop(0, NWIN, win, 0)


_p5 = pl.kernel(
    _p5_body,
    out_type=jax.ShapeDtypeStruct((E * 3,), jnp.float32),
    mesh=_MESH,
    compiler_params=pltpu.CompilerParams(needs_layout_passes=False, use_tc_tiling_on_sc=False),
    scratch_types=[
        pltpu.VMEM((NPAD,), jnp.int32),
        pltpu.VMEM((NPAIR * 3,), jnp.float32),
        pltpu.VMEM((WIN,), jnp.int32),
        pltpu.VMEM((WIN,), jnp.int32),
        pltpu.VMEM((WIN * 3,), jnp.float32),
    ],
)


def kernel(elem_idx, edge_index, distances, embed, codebook, fw1, fb1, fw2, fb2, mw1, mb1, mw2, mb2):
    src = edge_index[0]
    tgt = edge_index[1]
    elem_tgt = _p1(elem_idx, tgt)
    messages = _p2(
        distances, RBF_CENTERS.reshape(1, NRBF), elem_tgt, embed, fw1,
        fb1.reshape(1, D), fw2, fb2.reshape(1, D),
    )
    src2 = src.reshape(E // CH, CH)
    zeros = jnp.zeros((TPB, D), jnp.float32)
    agg = _p3(src2, messages, zeros)
    elem_pad = jnp.pad(elem_idx, (0, NPAD - N))
    atoms_pad, lossbuf, t_tab = _p4(
        elem_pad, agg, embed, codebook, codebook.T, mw1, mb1.reshape(1, D), mw2,
        mb2.reshape(1, 3),
    )
    params_flat = _p5(src, tgt, atoms_pad, t_tab.reshape(NPAIR * 3))
    return (params_flat.reshape(E, 3), atoms_pad[:N], lossbuf[0, 0])


# speedup vs baseline: 10.3159x; 1.1257x over previous
import jax
import jax.numpy as jnp
from jax import lax
from jax.experimental import pallas as pl
from jax.experimental.pallas import tpu as pltpu
from jax.experimental.pallas import tpu_sc as plsc

N = 100000
E = 3200000
NUM_ELEM = 5
D = 16
NRBF = 16
CODES_PER = 4
K = NUM_ELEM * CODES_PER
NPAIR = K * K
ANGSTROM_TO_BOHR = 1.8897259886
R_MIN = 0.5 * ANGSTROM_TO_BOHR
R_MAX = 7.56
RBF_CENTERS = jnp.linspace(R_MIN, R_MAX, NRBF, dtype=jnp.float32)
RBF_WIDTH = (R_MAX - R_MIN) / NRBF
COMMIT_W = 0.25

NC = 2
NS = 16
NW = NC * NS
EPW = E // NW
WIN = 2000
NWIN = EPW // WIN
CH = 80
G16 = WIN // 16
WIN3 = 800
NWIN3 = EPW // WIN3
NCHUNK3 = WIN3 // CH

NPAD = 102400
TPB = NPAD // NS
BN = 5120
NB = NPAD // BN
BE = 5120
NEB = E // BE

_MESH = plsc.VectorSubcoreMesh(
    core_axis_name="c", subcore_axis_name="s", num_cores=NC, num_subcores=NS
)


def _p1_body(elem_hbm, tgt_hbm, out_hbm, elem_v, tgt_v, out_v):
    wid = lax.axis_index("s") * NC + lax.axis_index("c")
    pltpu.sync_copy(elem_hbm, elem_v)

    def win(w, carry):
        base = wid * EPW + w * WIN
        pltpu.sync_copy(tgt_hbm.at[pl.ds(base, WIN)], tgt_v)

        def grp(j, carry2):
            idx = tgt_v[pl.ds(j * 16, 16)]
            out_v[pl.ds(j * 16, 16)] = plsc.load_gather(elem_v, [idx])
            return carry2

        lax.fori_loop(0, G16, grp, 0)
        pltpu.sync_copy(out_v, out_hbm.at[pl.ds(base, WIN)])
        return carry

    lax.fori_loop(0, NWIN, win, 0)


_p1 = pl.kernel(
    _p1_body,
    out_type=jax.ShapeDtypeStruct((E,), jnp.int32),
    mesh=_MESH,
    compiler_params=pltpu.CompilerParams(needs_layout_passes=False, use_tc_tiling_on_sc=False),
    scratch_types=[
        pltpu.VMEM((N,), jnp.int32),
        pltpu.VMEM((WIN,), jnp.int32),
        pltpu.VMEM((WIN,), jnp.int32),
    ],
)


def _p2_body(d_ref, ctr_ref, et_ref, emb_ref, fw1_ref, fb1_ref, fw2_ref, fb2_ref, out_ref):
    d = d_ref[:]
    z = (d[:, None] - ctr_ref[:]) / RBF_WIDTH
    rbf = jnp.exp(-0.5 * (z * z))
    h1 = jax.nn.silu(jnp.dot(rbf, fw1_ref[:], preferred_element_type=jnp.float32) + fb1_ref[:])
    wf = jnp.dot(h1, fw2_ref[:], preferred_element_type=jnp.float32) + fb2_ref[:]
    et = et_ref[:][:, None]
    e = emb_ref[:]
    row = jnp.where(
        et == 0,
        e[0][None, :],
        jnp.where(
            et == 1,
            e[1][None, :],
            jnp.where(et == 2, e[2][None, :], jnp.where(et == 3, e[3][None, :], e[4][None, :])),
        ),
    )
    out_ref[:] = row * wf


_p2 = pl.pallas_call(
    _p2_body,
    grid=(NEB,),
    in_specs=[
        pl.BlockSpec((BE,), lambda i: (i,)),
        pl.BlockSpec((1, NRBF), lambda i: (0, 0)),
        pl.BlockSpec((BE,), lambda i: (i,)),
        pl.BlockSpec((NUM_ELEM, D), lambda i: (0, 0)),
        pl.BlockSpec((NRBF, D), lambda i: (0, 0)),
        pl.BlockSpec((1, D), lambda i: (0, 0)),
        pl.BlockSpec((D, D), lambda i: (0, 0)),
        pl.BlockSpec((1, D), lambda i: (0, 0)),
    ],
    out_specs=pl.BlockSpec((BE, D), lambda i: (i, 0)),
    out_shape=jax.ShapeDtypeStruct((E, D), jnp.float32),
)


def _p3_body(src_hbm, msg_hbm, zer_hbm, agg_hbm, agg_s, idx_v, msg_v):
    c = lax.axis_index("c")
    s = lax.axis_index("s")
    wid = s * NC + c
    pltpu.sync_copy(zer_hbm, agg_s.at[pl.ds(s * TPB, TPB)])
    plsc.subcore_barrier()

    def win(w, carry):
        base = wid * EPW + w * WIN3
        pltpu.sync_copy(src_hbm.at[pl.ds(base, WIN3)], idx_v)
        pltpu.sync_copy(msg_hbm.at[pl.ds(base, WIN3)], msg_v)
        for j in range(NCHUNK3):
            pltpu.sync_copy(
                msg_v.at[pl.ds(j * CH, CH)],
                agg_s.at[idx_v.at[pl.ds(j * CH, CH)]],
                add=True,
            )
        return carry

    lax.fori_loop(0, NWIN3, win, 0)
    plsc.subcore_barrier()
    pltpu.sync_copy(agg_s.at[pl.ds(s * TPB, TPB)], agg_hbm.at[c, pl.ds(s * TPB, TPB)])


_p3 = pl.kernel(
    _p3_body,
    out_type=jax.ShapeDtypeStruct((NC, NPAD, D), jnp.float32),
    mesh=_MESH,
    compiler_params=pltpu.CompilerParams(needs_layout_passes=False, use_tc_tiling_on_sc=False),
    scratch_types=[
        pltpu.VMEM_SHARED((NPAD, D), jnp.float32),
        pltpu.VMEM((WIN3,), jnp.int32),
        pltpu.VMEM((WIN3, D), jnp.float32),
    ],
)


def _p4_body(
    elem_ref, agg_ref, emb_ref, cb_ref, cbT_ref, mw1_ref, mb1_ref, mw2_ref, mb2_ref,
    atom_ref, loss_ref, t_ref,
):
    i = pl.program_id(0)
    ei = elem_ref[:][:, None]
    e = emb_ref[:]
    h0 = jnp.where(
        ei == 0,
        e[0][None, :],
        jnp.where(
            ei == 1,
            e[1][None, :],
            jnp.where(ei == 2, e[2][None, :], jnp.where(ei == 3, e[3][None, :], e[4][None, :])),
        ),
    )
    h = h0 + agg_ref[0] + agg_ref[1]
    cbT = cbT_ref[:]
    xc = jnp.dot(h, cbT, preferred_element_type=jnp.float32)
    hsq = jnp.sum(h * h, axis=1, keepdims=True)
    csq = jnp.sum(cbT * cbT, axis=0, keepdims=True)
    d2 = jnp.maximum(hsq + csq - 2.0 * xc, 0.0)
    code_elem = lax.broadcasted_iota(jnp.int32, (1, K), 1) // CODES_PER
    masked = jnp.where(ei == code_elem, d2, jnp.inf)
    m = jnp.min(masked, axis=1)
    jio = lax.broadcasted_iota(jnp.int32, (BN, K), 1)
    atom_ref[:] = jnp.min(jnp.where(masked == m[:, None], jio, K), axis=1)
    rows = i * BN + lax.broadcasted_iota(jnp.int32, (BN,), 0)
    partial = jnp.sum(jnp.where(rows < N, m, 0.0)) * (COMMIT_W / (N * D))

    @pl.when(i == 0)
    def _init():
        loss_ref[:] = jnp.zeros((8, 128), jnp.float32)
        cb = cb_ref[:]
        ii = lax.broadcasted_iota(jnp.int32, (NPAIR, K), 1)
        ridx = lax.broadcasted_iota(jnp.int32, (NPAIR, 1), 0)
        oh = (ii == ridx // K).astype(jnp.float32) + (ii == ridx % K).astype(jnp.float32)
        pair = jnp.dot(oh, cb, preferred_element_type=jnp.float32)
        hh = jax.nn.silu(jnp.dot(pair, mw1_ref[:], preferred_element_type=jnp.float32) + mb1_ref[:])
        t_ref[:] = jnp.dot(hh, mw2_ref[:], preferred_element_type=jnp.float32) + mb2_ref[:]

    loss_ref[:] = loss_ref[:] + partial


_p4 = pl.pallas_call(
    _p4_body,
    grid=(NB,),
    in_specs=[
        pl.BlockSpec((BN,), lambda i: (i,)),
        pl.BlockSpec((NC, BN, D), lambda i: (0, i, 0)),
        pl.BlockSpec((NUM_ELEM, D), lambda i: (0, 0)),
        pl.BlockSpec((K, D), lambda i: (0, 0)),
        pl.BlockSpec((D, K), lambda i: (0, 0)),
        pl.BlockSpec((D, D), lambda i: (0, 0)),
        pl.BlockSpec((1, D), lambda i: (0, 0)),
        pl.BlockSpec((D, 3), lambda i: (0, 0)),
        pl.BlockSpec((1, 3), lambda i: (0, 0)),
    ],
    out_specs=[
        pl.BlockSpec((BN,), lambda i: (i,)),
        pl.BlockSpec((8, 128), lambda i: (0, 0)),
        pl.BlockSpec((NPAIR, 3), lambda i: (0, 0)),
    ],
    out_shape=[
        jax.ShapeDtypeStruct((NPAD,), jnp.int32),
        jax.ShapeDtypeStruct((8, 128), jnp.float32),
        jax.ShapeDtypeStruct((NPAIR, 3), jnp.float32),
    ],
)


def _p5_body(src_hbm, tgt_hbm, atoms_hbm, t_hbm, out_hbm, atoms_v, t_v, src_v, tgt_v, out_v):
    wid = lax.axis_index("s") * NC + lax.axis_index("c")
    pltpu.sync_copy(atoms_hbm, atoms_v)
    pltpu.sync_copy(t_hbm, t_v)
    arange16 = lax.broadcasted_iota(jnp.int32, (16,), 0)

    def win(w, carry):
        base = wid * EPW + w * WIN
        pltpu.sync_copy(src_hbm.at[pl.ds(base, WIN)], src_v)
        pltpu.sync_copy(tgt_hbm.at[pl.ds(base, WIN)], tgt_v)

        def grp(j, carry2):
            s16 = src_v[pl.ds(j * 16, 16)]
            t16 = tgt_v[pl.ds(j * 16, 16)]
            a = plsc.load_gather(atoms_v, [s16])
            b = plsc.load_gather(atoms_v, [t16])
            f3 = (a * K + b) * 3
            rows = arange16 + j * 16
            for p in range(3):
                plsc.store_scatter(
                    out_v, [rows, jnp.full((16,), p, jnp.int32)],
                    plsc.load_gather(t_v, [f3 + p]),
                )
            return carry2

        lax.fori_loop(0, G16, grp, 0)
        pltpu.sync_copy(out_v, out_hbm.at[pl.ds(base, WIN), :])
        return carry

    lax.fori_loop(0, NWIN, win, 0)


_p5 = pl.kernel(
    _p5_body,
    out_type=jax.ShapeDtypeStruct((E, 3), jnp.float32),
    mesh=_MESH,
    compiler_params=pltpu.CompilerParams(needs_layout_passes=False, use_tc_tiling_on_sc=False),
    scratch_types=[
        pltpu.VMEM((NPAD,), jnp.int32),
        pltpu.VMEM((NPAIR * 3,), jnp.float32),
        pltpu.VMEM((WIN,), jnp.int32),
        pltpu.VMEM((WIN,), jnp.int32),
        pltpu.VMEM((WIN, 3), jnp.float32),
    ],
)


def kernel(elem_idx, edge_index, distances, embed, codebook, fw1, fb1, fw2, fb2, mw1, mb1, mw2, mb2):
    src = edge_index[0]
    tgt = edge_index[1]
    elem_tgt = _p1(elem_idx, tgt)
    messages = _p2(
        distances, RBF_CENTERS.reshape(1, NRBF), elem_tgt, embed, fw1,
        fb1.reshape(1, D), fw2, fb2.reshape(1, D),
    )
    zeros = jnp.zeros((TPB, D), jnp.float32)
    agg = _p3(src, messages, zeros)
    elem_pad = jnp.pad(elem_idx, (0, NPAD - N))
    atoms_pad, lossbuf, t_tab = _p4(
        elem_pad, agg, embed, codebook, codebook.T, mw1, mb1.reshape(1, D), mw2,
        mb2.reshape(1, 3),
    )
    params = _p5(src, tgt, atoms_pad, t_tab.reshape(NPAIR * 3))
    return (params, atoms_pad[:N], lossbuf[0, 0])


# speedup vs baseline: 15.1831x; 1.4718x over previous
import jax
import jax.numpy as jnp
from jax import lax
from jax.experimental import pallas as pl
from jax.experimental.pallas import tpu as pltpu
from jax.experimental.pallas import tpu_sc as plsc

N = 100000
E = 3200000
NUM_ELEM = 5
D = 16
NRBF = 16
CODES_PER = 4
K = NUM_ELEM * CODES_PER
NPAIR = K * K
ANGSTROM_TO_BOHR = 1.8897259886
R_MIN = 0.5 * ANGSTROM_TO_BOHR
R_MAX = 7.56
RBF_CENTERS = jnp.linspace(R_MIN, R_MAX, NRBF, dtype=jnp.float32)
RBF_WIDTH = (R_MAX - R_MIN) / NRBF
COMMIT_W = 0.25

NC = 2
NS = 16
NW = NC * NS
EPW = E // NW
WIN = 2000
NWIN = EPW // WIN
CH = 80
G16 = WIN // 16
WIN3 = 800
NWIN3 = EPW // WIN3
NCHUNK3 = WIN3 // CH

NPAD = 102400
TPB = NPAD // NS
BN = 5120
NB = NPAD // BN
BE = 5120
NEB = E // BE

_MESH = plsc.VectorSubcoreMesh(
    core_axis_name="c", subcore_axis_name="s", num_cores=NC, num_subcores=NS
)


def _p1_body(elem_hbm, tgt_hbm, out_hbm, elem_v, tgt_v, out_v):
    wid = lax.axis_index("s") * NC + lax.axis_index("c")
    pltpu.sync_copy(elem_hbm, elem_v)

    def win(w, carry):
        base = wid * EPW + w * WIN
        pltpu.sync_copy(tgt_hbm.at[pl.ds(base, WIN)], tgt_v)

        def grp(j, carry2):
            idx = tgt_v[pl.ds(j * 16, 16)]
            out_v[pl.ds(j * 16, 16)] = plsc.load_gather(elem_v, [idx])
            return carry2

        lax.fori_loop(0, G16, grp, 0)
        pltpu.sync_copy(out_v, out_hbm.at[pl.ds(base, WIN)])
        return carry

    lax.fori_loop(0, NWIN, win, 0)


_p1 = pl.kernel(
    _p1_body,
    out_type=jax.ShapeDtypeStruct((E,), jnp.int32),
    mesh=_MESH,
    compiler_params=pltpu.CompilerParams(needs_layout_passes=False, use_tc_tiling_on_sc=False),
    scratch_types=[
        pltpu.VMEM((N,), jnp.int32),
        pltpu.VMEM((WIN,), jnp.int32),
        pltpu.VMEM((WIN,), jnp.int32),
    ],
)


def _p2_body(d_ref, ctr_ref, et_ref, emb_ref, fw1_ref, fb1_ref, fw2_ref, fb2_ref, out_ref):
    d = d_ref[:]
    z = (d[:, None] - ctr_ref[:]) / RBF_WIDTH
    rbf = jnp.exp(-0.5 * (z * z))
    h1 = jax.nn.silu(jnp.dot(rbf, fw1_ref[:], preferred_element_type=jnp.float32) + fb1_ref[:])
    wf = jnp.dot(h1, fw2_ref[:], preferred_element_type=jnp.float32) + fb2_ref[:]
    et = et_ref[:][:, None]
    e = emb_ref[:]
    row = jnp.where(
        et == 0,
        e[0][None, :],
        jnp.where(
            et == 1,
            e[1][None, :],
            jnp.where(et == 2, e[2][None, :], jnp.where(et == 3, e[3][None, :], e[4][None, :])),
        ),
    )
    out_ref[:] = row * wf


_p2 = pl.pallas_call(
    _p2_body,
    grid=(NEB,),
    in_specs=[
        pl.BlockSpec((BE,), lambda i: (i,)),
        pl.BlockSpec((1, NRBF), lambda i: (0, 0)),
        pl.BlockSpec((BE,), lambda i: (i,)),
        pl.BlockSpec((NUM_ELEM, D), lambda i: (0, 0)),
        pl.BlockSpec((NRBF, D), lambda i: (0, 0)),
        pl.BlockSpec((1, D), lambda i: (0, 0)),
        pl.BlockSpec((D, D), lambda i: (0, 0)),
        pl.BlockSpec((1, D), lambda i: (0, 0)),
    ],
    out_specs=pl.BlockSpec((BE, D), lambda i: (i, 0)),
    out_shape=jax.ShapeDtypeStruct((E, D), jnp.float32),
)


def _p3_body(src_hbm, msg_hbm, zer_hbm, agg_hbm, agg_s, idx_v, msg_v):
    c = lax.axis_index("c")
    s = lax.axis_index("s")
    wid = s * NC + c
    pltpu.sync_copy(zer_hbm, agg_s.at[pl.ds(s * TPB, TPB)])
    plsc.subcore_barrier()

    def win(w, carry):
        base = wid * EPW + w * WIN3
        pltpu.sync_copy(src_hbm.at[pl.ds(base, WIN3)], idx_v)
        pltpu.sync_copy(msg_hbm.at[pl.ds(base, WIN3)], msg_v)
        for j in range(NCHUNK3):
            pltpu.sync_copy(
                msg_v.at[pl.ds(j * CH, CH)],
                agg_s.at[idx_v.at[pl.ds(j * CH, CH)]],
                add=True,
            )
        return carry

    lax.fori_loop(0, NWIN3, win, 0)
    plsc.subcore_barrier()
    pltpu.sync_copy(agg_s.at[pl.ds(s * TPB, TPB)], agg_hbm.at[c, pl.ds(s * TPB, TPB)])


_p3 = pl.kernel(
    _p3_body,
    out_type=jax.ShapeDtypeStruct((NC, NPAD, D), jnp.float32),
    mesh=_MESH,
    compiler_params=pltpu.CompilerParams(needs_layout_passes=False, use_tc_tiling_on_sc=False),
    scratch_types=[
        pltpu.VMEM_SHARED((NPAD, D), jnp.float32),
        pltpu.VMEM((WIN3,), jnp.int32),
        pltpu.VMEM((WIN3, D), jnp.float32),
    ],
)


def _p4_body(
    elem_ref, agg_ref, emb_ref, cb_ref, cbT_ref, mw1_ref, mb1_ref, mw2_ref, mb2_ref,
    atom_ref, loss_ref, t_ref,
):
    i = pl.program_id(0)
    ei = elem_ref[:][:, None]
    e = emb_ref[:]
    h0 = jnp.where(
        ei == 0,
        e[0][None, :],
        jnp.where(
            ei == 1,
            e[1][None, :],
            jnp.where(ei == 2, e[2][None, :], jnp.where(ei == 3, e[3][None, :], e[4][None, :])),
        ),
    )
    h = h0 + agg_ref[0] + agg_ref[1]
    cbT = cbT_ref[:]
    xc = jnp.dot(h, cbT, preferred_element_type=jnp.float32)
    hsq = jnp.sum(h * h, axis=1, keepdims=True)
    csq = jnp.sum(cbT * cbT, axis=0, keepdims=True)
    d2 = jnp.maximum(hsq + csq - 2.0 * xc, 0.0)
    code_elem = lax.broadcasted_iota(jnp.int32, (1, K), 1) // CODES_PER
    masked = jnp.where(ei == code_elem, d2, jnp.inf)
    m = jnp.min(masked, axis=1)
    jio = lax.broadcasted_iota(jnp.int32, (BN, K), 1)
    atom_ref[:] = jnp.min(jnp.where(masked == m[:, None], jio, K), axis=1)
    rows = i * BN + lax.broadcasted_iota(jnp.int32, (BN,), 0)
    partial = jnp.sum(jnp.where(rows < N, m, 0.0)) * (COMMIT_W / (N * D))

    @pl.when(i == 0)
    def _init():
        loss_ref[:] = jnp.zeros((8, 128), jnp.float32)
        cb = cb_ref[:]
        ii = lax.broadcasted_iota(jnp.int32, (NPAIR, K), 1)
        ridx = lax.broadcasted_iota(jnp.int32, (NPAIR, 1), 0)
        oh = (ii == ridx // K).astype(jnp.float32) + (ii == ridx % K).astype(jnp.float32)
        pair = jnp.dot(oh, cb, preferred_element_type=jnp.float32)
        hh = jax.nn.silu(jnp.dot(pair, mw1_ref[:], preferred_element_type=jnp.float32) + mb1_ref[:])
        t_ref[:] = jnp.dot(hh, mw2_ref[:], preferred_element_type=jnp.float32) + mb2_ref[:]

    loss_ref[:] = loss_ref[:] + partial


_p4 = pl.pallas_call(
    _p4_body,
    grid=(NB,),
    in_specs=[
        pl.BlockSpec((BN,), lambda i: (i,)),
        pl.BlockSpec((NC, BN, D), lambda i: (0, i, 0)),
        pl.BlockSpec((NUM_ELEM, D), lambda i: (0, 0)),
        pl.BlockSpec((K, D), lambda i: (0, 0)),
        pl.BlockSpec((D, K), lambda i: (0, 0)),
        pl.BlockSpec((D, D), lambda i: (0, 0)),
        pl.BlockSpec((1, D), lambda i: (0, 0)),
        pl.BlockSpec((D, 3), lambda i: (0, 0)),
        pl.BlockSpec((1, 3), lambda i: (0, 0)),
    ],
    out_specs=[
        pl.BlockSpec((BN,), lambda i: (i,)),
        pl.BlockSpec((8, 128), lambda i: (0, 0)),
        pl.BlockSpec((NPAIR, 3), lambda i: (0, 0)),
    ],
    out_shape=[
        jax.ShapeDtypeStruct((NPAD,), jnp.int32),
        jax.ShapeDtypeStruct((8, 128), jnp.float32),
        jax.ShapeDtypeStruct((NPAIR, 3), jnp.float32),
    ],
)


def _p5_body(src_hbm, tgt_hbm, atoms_hbm, t_hbm, out0_hbm, out1_hbm, out2_hbm,
             atoms_v, t_v, src_v, tgt_v, o0_v, o1_v, o2_v):
    wid = lax.axis_index("s") * NC + lax.axis_index("c")
    pltpu.sync_copy(atoms_hbm, atoms_v)
    pltpu.sync_copy(t_hbm, t_v)

    def win(w, carry):
        base = wid * EPW + w * WIN
        pltpu.sync_copy(src_hbm.at[pl.ds(base, WIN)], src_v)
        pltpu.sync_copy(tgt_hbm.at[pl.ds(base, WIN)], tgt_v)

        def grp(j, carry2):
            s16 = src_v[pl.ds(j * 16, 16)]
            t16 = tgt_v[pl.ds(j * 16, 16)]
            a = plsc.load_gather(atoms_v, [s16])
            b = plsc.load_gather(atoms_v, [t16])
            f3 = (a * K + b) * 3
            o0_v[pl.ds(j * 16, 16)] = plsc.load_gather(t_v, [f3])
            o1_v[pl.ds(j * 16, 16)] = plsc.load_gather(t_v, [f3 + 1])
            o2_v[pl.ds(j * 16, 16)] = plsc.load_gather(t_v, [f3 + 2])
            return carry2

        lax.fori_loop(0, G16, grp, 0)
        pltpu.sync_copy(o0_v, out0_hbm.at[pl.ds(base, WIN)])
        pltpu.sync_copy(o1_v, out1_hbm.at[pl.ds(base, WIN)])
        pltpu.sync_copy(o2_v, out2_hbm.at[pl.ds(base, WIN)])
        return carry

    lax.fori_loop(0, NWIN, win, 0)


_p5 = pl.kernel(
    _p5_body,
    out_type=[
        jax.ShapeDtypeStruct((E,), jnp.float32),
        jax.ShapeDtypeStruct((E,), jnp.float32),
        jax.ShapeDtypeStruct((E,), jnp.float32),
    ],
    mesh=_MESH,
    compiler_params=pltpu.CompilerParams(needs_layout_passes=False, use_tc_tiling_on_sc=False),
    scratch_types=[
        pltpu.VMEM((NPAD,), jnp.int32),
        pltpu.VMEM((NPAIR * 3,), jnp.float32),
        pltpu.VMEM((WIN,), jnp.int32),
        pltpu.VMEM((WIN,), jnp.int32),
        pltpu.VMEM((WIN,), jnp.float32),
        pltpu.VMEM((WIN,), jnp.float32),
        pltpu.VMEM((WIN,), jnp.float32),
    ],
)


def kernel(elem_idx, edge_index, distances, embed, codebook, fw1, fb1, fw2, fb2, mw1, mb1, mw2, mb2):
    src = edge_index[0]
    tgt = edge_index[1]
    elem_tgt = _p1(elem_idx, tgt)
    messages = _p2(
        distances, RBF_CENTERS.reshape(1, NRBF), elem_tgt, embed, fw1,
        fb1.reshape(1, D), fw2, fb2.reshape(1, D),
    )
    zeros = jnp.zeros((TPB, D), jnp.float32)
    agg = _p3(src, messages, zeros)
    elem_pad = jnp.pad(elem_idx, (0, NPAD - N))
    atoms_pad, lossbuf, t_tab = _p4(
        elem_pad, agg, embed, codebook, codebook.T, mw1, mb1.reshape(1, D), mw2,
        mb2.reshape(1, 3),
    )
    p0, p1, p2 = _p5(src, tgt, atoms_pad, t_tab.reshape(NPAIR * 3))
    params = jnp.transpose(jnp.stack([p0, p1, p2], axis=0), (1, 0))
    return (params, atoms_pad[:N], lossbuf[0, 0])


# speedup vs baseline: 21.1521x; 1.3931x over previous
import jax
import jax.numpy as jnp
from jax import lax
from jax.experimental import pallas as pl
from jax.experimental.pallas import tpu as pltpu
from jax.experimental.pallas import tpu_sc as plsc

N = 100000
E = 3200000
NUM_ELEM = 5
D = 16
NRBF = 16
CODES_PER = 4
K = NUM_ELEM * CODES_PER
NPAIR = K * K
ANGSTROM_TO_BOHR = 1.8897259886
R_MIN = 0.5 * ANGSTROM_TO_BOHR
R_MAX = 7.56
RBF_CENTERS = jnp.linspace(R_MIN, R_MAX, NRBF, dtype=jnp.float32)
RBF_WIDTH = (R_MAX - R_MIN) / NRBF
COMMIT_W = 0.25

NC = 2
NS = 16
NW = NC * NS
EPW = E // NW
WIN = 2000
NWIN = EPW // WIN
CH = 80
G16 = WIN // 16
WIN3 = 400
NWIN3 = EPW // WIN3
NCHUNK3 = WIN3 // CH

NPAD = 102400
TPB = NPAD // NS
BN = 5120
NB = NPAD // BN
BE = 5120
NEB = E // BE

_MESH = plsc.VectorSubcoreMesh(
    core_axis_name="c", subcore_axis_name="s", num_cores=NC, num_subcores=NS
)


def _p1_body(elem_hbm, tgt_hbm, out_hbm, elem_v, tgt_v, out_v):
    wid = lax.axis_index("s") * NC + lax.axis_index("c")
    pltpu.sync_copy(elem_hbm, elem_v)

    def win(w, carry):
        base = wid * EPW + w * WIN
        pltpu.sync_copy(tgt_hbm.at[pl.ds(base, WIN)], tgt_v)

        def grp(j, carry2):
            idx = tgt_v[pl.ds(j * 16, 16)]
            out_v[pl.ds(j * 16, 16)] = plsc.load_gather(elem_v, [idx])
            return carry2

        lax.fori_loop(0, G16, grp, 0)
        pltpu.sync_copy(out_v, out_hbm.at[pl.ds(base, WIN)])
        return carry

    lax.fori_loop(0, NWIN, win, 0)


_p1 = pl.kernel(
    _p1_body,
    out_type=jax.ShapeDtypeStruct((E,), jnp.int32),
    mesh=_MESH,
    compiler_params=pltpu.CompilerParams(needs_layout_passes=False, use_tc_tiling_on_sc=False),
    scratch_types=[
        pltpu.VMEM((N,), jnp.int32),
        pltpu.VMEM((WIN,), jnp.int32),
        pltpu.VMEM((WIN,), jnp.int32),
    ],
)


def _p2_body(d_ref, ctr_ref, et_ref, embT_ref, fw1T_ref, fb1_ref, fw2T_ref, fb2_ref, *out_refs):
    d = d_ref[:].reshape(1, BE)
    z = (d - ctr_ref[:]) / RBF_WIDTH
    rbf = jnp.exp(-0.5 * (z * z))
    h1 = jax.nn.silu(
        jnp.dot(fw1T_ref[:], rbf, preferred_element_type=jnp.float32) + fb1_ref[:]
    )
    wf = jnp.dot(fw2T_ref[:], h1, preferred_element_type=jnp.float32) + fb2_ref[:]
    et = et_ref[:].reshape(1, BE)
    e = embT_ref[:]
    row = jnp.where(
        et == 0,
        e[:, 0:1],
        jnp.where(
            et == 1,
            e[:, 1:2],
            jnp.where(et == 2, e[:, 2:3], jnp.where(et == 3, e[:, 3:4], e[:, 4:5])),
        ),
    )
    msgs = row * wf
    for k in range(D):
        out_refs[k][:] = msgs[k : k + 1, :].reshape(BE)


_p2 = pl.pallas_call(
    _p2_body,
    grid=(NEB,),
    in_specs=[
        pl.BlockSpec((BE,), lambda i: (i,)),
        pl.BlockSpec((NRBF, 1), lambda i: (0, 0)),
        pl.BlockSpec((BE,), lambda i: (i,)),
        pl.BlockSpec((D, NUM_ELEM), lambda i: (0, 0)),
        pl.BlockSpec((D, NRBF), lambda i: (0, 0)),
        pl.BlockSpec((D, 1), lambda i: (0, 0)),
        pl.BlockSpec((D, D), lambda i: (0, 0)),
        pl.BlockSpec((D, 1), lambda i: (0, 0)),
    ],
    out_specs=[pl.BlockSpec((BE,), lambda i: (i,)) for _ in range(D)],
    out_shape=[jax.ShapeDtypeStruct((E,), jnp.float32) for _ in range(D)],
)


def _p3_body(src_hbm, *rest):
    msg_hbms = rest[:D]
    zer_hbm = rest[D]
    agg_hbm = rest[D + 1]
    agg_s, idx_v, msgt_v, msg_v, sem = rest[D + 2 :]
    c = lax.axis_index("c")
    s = lax.axis_index("s")
    wid = s * NC + c
    pltpu.sync_copy(zer_hbm, agg_s.at[pl.ds(s * TPB, TPB)])
    plsc.subcore_barrier()
    iotaw = lax.broadcasted_iota(jnp.int32, (16,), 0) * WIN3

    def win(w, carry):
        base = wid * EPW + w * WIN3
        pltpu.sync_copy(src_hbm.at[pl.ds(base, WIN3)], idx_v)
        descs = []
        for k in range(D):
            descs.append(
                pltpu.async_copy(
                    msg_hbms[k].at[pl.ds(base, WIN3)],
                    msgt_v.at[pl.ds(k * WIN3, WIN3)],
                    sem,
                )
            )
        for d_ in descs:
            d_.wait()

        def tr(e, carry2):
            msg_v[e, :] = plsc.load_gather(msgt_v, [iotaw + e])
            return carry2

        lax.fori_loop(0, WIN3, tr, 0)
        for j in range(NCHUNK3):
            pltpu.sync_copy(
                msg_v.at[pl.ds(j * CH, CH)],
                agg_s.at[idx_v.at[pl.ds(j * CH, CH)]],
                add=True,
            )
        return carry

    lax.fori_loop(0, NWIN3, win, 0)
    plsc.subcore_barrier()
    pltpu.sync_copy(agg_s.at[pl.ds(s * TPB, TPB)], agg_hbm.at[c, pl.ds(s * TPB, TPB)])


_p3 = pl.kernel(
    _p3_body,
    out_type=jax.ShapeDtypeStruct((NC, NPAD, D), jnp.float32),
    mesh=_MESH,
    compiler_params=pltpu.CompilerParams(needs_layout_passes=False, use_tc_tiling_on_sc=False),
    scratch_types=[
        pltpu.VMEM_SHARED((NPAD, D), jnp.float32),
        pltpu.VMEM((WIN3,), jnp.int32),
        pltpu.VMEM((WIN3 * D,), jnp.float32),
        pltpu.VMEM((WIN3, D), jnp.float32),
        pltpu.SemaphoreType.DMA,
    ],
)


def _p4_body(
    elem_ref, agg_ref, emb_ref, cb_ref, cbT_ref, mw1_ref, mb1_ref, mw2_ref, mb2_ref,
    atom_ref, loss_ref, t_ref,
):
    i = pl.program_id(0)
    ei = elem_ref[:][:, None]
    e = emb_ref[:]
    h0 = jnp.where(
        ei == 0,
        e[0][None, :],
        jnp.where(
            ei == 1,
            e[1][None, :],
            jnp.where(ei == 2, e[2][None, :], jnp.where(ei == 3, e[3][None, :], e[4][None, :])),
        ),
    )
    h = h0 + agg_ref[0] + agg_ref[1]
    cbT = cbT_ref[:]
    xc = jnp.dot(h, cbT, preferred_element_type=jnp.float32)
    hsq = jnp.sum(h * h, axis=1, keepdims=True)
    csq = jnp.sum(cbT * cbT, axis=0, keepdims=True)
    d2 = jnp.maximum(hsq + csq - 2.0 * xc, 0.0)
    code_elem = lax.broadcasted_iota(jnp.int32, (1, K), 1) // CODES_PER
    masked = jnp.where(ei == code_elem, d2, jnp.inf)
    m = jnp.min(masked, axis=1)
    jio = lax.broadcasted_iota(jnp.int32, (BN, K), 1)
    atom_ref[:] = jnp.min(jnp.where(masked == m[:, None], jio, K), axis=1)
    rows = i * BN + lax.broadcasted_iota(jnp.int32, (BN,), 0)
    partial = jnp.sum(jnp.where(rows < N, m, 0.0)) * (COMMIT_W / (N * D))

    @pl.when(i == 0)
    def _init():
        loss_ref[:] = jnp.zeros((8, 128), jnp.float32)
        cb = cb_ref[:]
        ii = lax.broadcasted_iota(jnp.int32, (NPAIR, K), 1)
        ridx = lax.broadcasted_iota(jnp.int32, (NPAIR, 1), 0)
        oh = (ii == ridx // K).astype(jnp.float32) + (ii == ridx % K).astype(jnp.float32)
        pair = jnp.dot(oh, cb, preferred_element_type=jnp.float32)
        hh = jax.nn.silu(jnp.dot(pair, mw1_ref[:], preferred_element_type=jnp.float32) + mb1_ref[:])
        t_ref[:] = jnp.dot(hh, mw2_ref[:], preferred_element_type=jnp.float32) + mb2_ref[:]

    loss_ref[:] = loss_ref[:] + partial


_p4 = pl.pallas_call(
    _p4_body,
    grid=(NB,),
    in_specs=[
        pl.BlockSpec((BN,), lambda i: (i,)),
        pl.BlockSpec((NC, BN, D), lambda i: (0, i, 0)),
        pl.BlockSpec((NUM_ELEM, D), lambda i: (0, 0)),
        pl.BlockSpec((K, D), lambda i: (0, 0)),
        pl.BlockSpec((D, K), lambda i: (0, 0)),
        pl.BlockSpec((D, D), lambda i: (0, 0)),
        pl.BlockSpec((1, D), lambda i: (0, 0)),
        pl.BlockSpec((D, 3), lambda i: (0, 0)),
        pl.BlockSpec((1, 3), lambda i: (0, 0)),
    ],
    out_specs=[
        pl.BlockSpec((BN,), lambda i: (i,)),
        pl.BlockSpec((8, 128), lambda i: (0, 0)),
        pl.BlockSpec((NPAIR, 3), lambda i: (0, 0)),
    ],
    out_shape=[
        jax.ShapeDtypeStruct((NPAD,), jnp.int32),
        jax.ShapeDtypeStruct((8, 128), jnp.float32),
        jax.ShapeDtypeStruct((NPAIR, 3), jnp.float32),
    ],
)


def _p5_body(src_hbm, tgt_hbm, atoms_hbm, t_hbm, out0_hbm, out1_hbm, out2_hbm,
             atoms_v, t_v, src_v, tgt_v, o0_v, o1_v, o2_v):
    wid = lax.axis_index("s") * NC + lax.axis_index("c")
    pltpu.sync_copy(atoms_hbm, atoms_v)
    pltpu.sync_copy(t_hbm, t_v)

    def win(w, carry):
        base = wid * EPW + w * WIN
        pltpu.sync_copy(src_hbm.at[pl.ds(base, WIN)], src_v)
        pltpu.sync_copy(tgt_hbm.at[pl.ds(base, WIN)], tgt_v)

        def grp(j, carry2):
            s16 = src_v[pl.ds(j * 16, 16)]
            t16 = tgt_v[pl.ds(j * 16, 16)]
            a = plsc.load_gather(atoms_v, [s16])
            b = plsc.load_gather(atoms_v, [t16])
            f3 = (a * K + b) * 3
            o0_v[pl.ds(j * 16, 16)] = plsc.load_gather(t_v, [f3])
            o1_v[pl.ds(j * 16, 16)] = plsc.load_gather(t_v, [f3 + 1])
            o2_v[pl.ds(j * 16, 16)] = plsc.load_gather(t_v, [f3 + 2])
            return carry2

        lax.fori_loop(0, G16, grp, 0)
        pltpu.sync_copy(o0_v, out0_hbm.at[pl.ds(base, WIN)])
        pltpu.sync_copy(o1_v, out1_hbm.at[pl.ds(base, WIN)])
        pltpu.sync_copy(o2_v, out2_hbm.at[pl.ds(base, WIN)])
        return carry

    lax.fori_loop(0, NWIN, win, 0)


_p5 = pl.kernel(
    _p5_body,
    out_type=[
        jax.ShapeDtypeStruct((E,), jnp.float32),
        jax.ShapeDtypeStruct((E,), jnp.float32),
        jax.ShapeDtypeStruct((E,), jnp.float32),
    ],
    mesh=_MESH,
    compiler_params=pltpu.CompilerParams(needs_layout_passes=False, use_tc_tiling_on_sc=False),
    scratch_types=[
        pltpu.VMEM((NPAD,), jnp.int32),
        pltpu.VMEM((NPAIR * 3,), jnp.float32),
        pltpu.VMEM((WIN,), jnp.int32),
        pltpu.VMEM((WIN,), jnp.int32),
        pltpu.VMEM((WIN,), jnp.float32),
        pltpu.VMEM((WIN,), jnp.float32),
        pltpu.VMEM((WIN,), jnp.float32),
    ],
)


def kernel(elem_idx, edge_index, distances, embed, codebook, fw1, fb1, fw2, fb2, mw1, mb1, mw2, mb2):
    src = edge_index[0]
    tgt = edge_index[1]
    elem_tgt = _p1(elem_idx, tgt)
    msg_planes = _p2(
        distances, RBF_CENTERS.reshape(NRBF, 1), elem_tgt, embed.T, fw1.T,
        fb1.reshape(D, 1), fw2.T, fb2.reshape(D, 1),
    )
    zeros = jnp.zeros((TPB, D), jnp.float32)
    agg = _p3(src, *msg_planes, zeros)
    elem_pad = jnp.pad(elem_idx, (0, NPAD - N))
    atoms_pad, lossbuf, t_tab = _p4(
        elem_pad, agg, embed, codebook, codebook.T, mw1, mb1.reshape(1, D), mw2,
        mb2.reshape(1, 3),
    )
    p0, p1, p2 = _p5(src, tgt, atoms_pad, t_tab.reshape(NPAIR * 3))
    params = jnp.transpose(jnp.stack([p0, p1, p2], axis=0), (1, 0))
    return (params, atoms_pad[:N], lossbuf[0, 0])
